# baseline (speedup 1.0000x reference)
import jax
import jax.numpy as jnp
from jax.experimental import pallas as pl
from jax.experimental.pallas import tpu as pltpu
from jax.experimental.pallas import tpu_sc as plsc

B, T, V, D, H, O = 128, 50, 1000, 128, 256, 1000
_GATHER_WINDOW = 128
_BT = 64
_UNROLL = 5
_TH = T // 2
_NS = _TH // _UNROLL
_F32 = jnp.float32
_BF16 = jnp.bfloat16


def _sc_gather(table, idx_flat):
    n = idx_flat.shape[0]
    idx2 = idx_flat.reshape(1, n)
    mesh = plsc.VectorSubcoreMesh(core_axis_name="core", subcore_axis_name="subcore")

    @pl.kernel(out_type=jax.ShapeDtypeStruct((n, D), table.dtype), mesh=mesh)
    def gather_kernel(tab_hbm, i_hbm, o_hbm):
        def body(i_vmem, o_vmem):
            pltpu.sync_copy(tab_hbm.at[i_vmem.at[0]], o_vmem)

        pltpu.emit_pipeline(
            body,
            grid=(n // _GATHER_WINDOW,),
            in_specs=[pl.BlockSpec((1, _GATHER_WINDOW), index_map=lambda i: (0, i))],
            out_specs=[pl.BlockSpec((_GATHER_WINDOW, D), index_map=lambda i: (i, 0))],
            core_axis_name=("core", "subcore"),
            dimension_semantics=(pltpu.PARALLEL,),
        )(i_hbm, o_hbm)

    return gather_kernel(table, idx2)


def _dot_t(x, w):
    return jax.lax.dot_general(x, w, (((1,), (1,)), ((), ())),
                               preferred_element_type=_F32)


def _body(emb_ref, hp_ref, cp_ref, wi0_ref, wh0_ref, b0_ref,
          wi1_ref, wh1_ref, b1_ref,
          out1_ref, hT_ref, cT_ref, h0s, c0s, h1s, c1s):
    t = pl.program_id(0)

    @pl.when(t == 0)
    def _():
        h0s[...] = hp_ref[0]
        c0s[...] = cp_ref[0]
        h1s[...] = hp_ref[1]
        c1s[...] = cp_ref[1]

    def gates_apply(g, cs, hs):
        i = jax.nn.sigmoid(g[:, 0 * H:1 * H])
        f = jax.nn.sigmoid(g[:, 1 * H:2 * H])
        gg = jnp.tanh(g[:, 2 * H:3 * H])
        o = jax.nn.sigmoid(g[:, 3 * H:4 * H])
        c_new = f * cs[...] + i * gg
        h_new = o * jnp.tanh(c_new)
        cs[...] = c_new
        hs[...] = h_new
        return h_new

    xb = emb_ref[...].reshape(_UNROLL * B, D).astype(_BF16)
    g0all = _dot_t(xb, wi0_ref[...]) + b0_ref[...]

    for k in range(_UNROLL):
        g0 = g0all[k * B:(k + 1) * B]
        g0 = g0 + _dot_t(h0s[...].astype(_BF16), wh0_ref[...])
        h0n = gates_apply(g0, c0s, h0s)
        g1 = _dot_t(h0n.astype(_BF16), wi1_ref[...])
        g1 = g1 + _dot_t(h1s[...].astype(_BF16), wh1_ref[...])
        g1 = g1 + b1_ref[...]
        out1_ref[k] = gates_apply(g1, c1s, h1s).astype(_BF16)

    @pl.when(t == _NS - 1)
    def _():
        hT_ref[0] = h0s[...]
        hT_ref[1] = h1s[...]
        cT_ref[0] = c0s[...]
        cT_ref[1] = c1s[...]


def _lstm_scan(emb_tb, h_prev, c_prev, wi0, wh0, b0, wi1, wh1, b1):
    full = lambda shape: pl.BlockSpec(shape, lambda t: tuple(0 for _ in shape))
    return pl.pallas_call(
        _body,
        grid=(_NS,),
        in_specs=[
            pl.BlockSpec((_UNROLL, B, D), lambda t: (t, 0, 0)),
            full((2, B, H)),
            full((2, B, H)),
            full((4 * H, D)),
            full((4 * H, H)),
            full((1, 4 * H)),
            full((4 * H, H)),
            full((4 * H, H)),
            full((1, 4 * H)),
        ],
        out_specs=[
            pl.BlockSpec((_UNROLL, B, H), lambda t: (t, 0, 0)),
            full((2, B, H)),
            full((2, B, H)),
        ],
        out_shape=[
            jax.ShapeDtypeStruct((_TH, B, H), _BF16),
            jax.ShapeDtypeStruct((2, B, H), _F32),
            jax.ShapeDtypeStruct((2, B, H), _F32),
        ],
        scratch_shapes=[pltpu.VMEM((B, H), _F32) for _ in range(4)],
    )(emb_tb, h_prev, c_prev, wi0, wh0, b0, wi1, wh1, b1)


def _fc_body(o1a_ref, o1b_ref, fw_ref, fb_ref, logT_ref, y2s):
    y2s[:, 0:_TH, :] = jnp.swapaxes(o1a_ref[...], 0, 1)
    y2s[:, _TH:T, :] = jnp.swapaxes(o1b_ref[...], 0, 1)
    y2 = y2s[...].reshape(_BT * T, H)
    z = jax.lax.dot_general(fw_ref[...], y2, (((1,), (1,)), ((), ())),
                            preferred_element_type=_F32)
    logT_ref[...] = z + fb_ref[...]


def _fc(out1a, out1b, fw, fb):
    return pl.pallas_call(
        _fc_body,
        grid=(B // _BT,),
        in_specs=[
            pl.BlockSpec((_TH, _BT, H), lambda i: (0, i, 0)),
            pl.BlockSpec((_TH, _BT, H), lambda i: (0, i, 0)),
            pl.BlockSpec((O, H), lambda i: (0, 0)),
            pl.BlockSpec((O, 1), lambda i: (0, 0)),
        ],
        out_specs=pl.BlockSpec((O, _BT * T), lambda i: (0, i)),
        out_shape=jax.ShapeDtypeStruct((O, B * T), _F32),
        scratch_shapes=[pltpu.VMEM((_BT, T, H), _BF16)],
    )(out1a, out1b, fw, fb)


def kernel(x, h_prev, c_prev, emb_table, W_ih0, W_hh0, b_ih0, b_hh0,
           W_ih1, W_hh1, b_ih1, b_hh1, fc_W, fc_b):
    idx = x.T.reshape(-1).astype(jnp.int32)
    emb_a = _sc_gather(emb_table, idx[:_TH * B]).reshape(_TH, B, D)
    emb_b = _sc_gather(emb_table, idx[_TH * B:]).reshape(_TH, B, D)
    b0 = (b_ih0 + b_hh0).reshape(1, 4 * H)
    b1 = (b_ih1 + b_hh1).reshape(1, 4 * H)
    ws = (W_ih0.astype(_BF16), W_hh0.astype(_BF16), b0,
          W_ih1.astype(_BF16), W_hh1.astype(_BF16), b1)
    out1a, hA, cA = _lstm_scan(emb_a, h_prev, c_prev, *ws)
    out1b, hT, cT = _lstm_scan(emb_b, hA, cA, *ws)
    logT = _fc(out1a, out1b, fc_W.astype(_BF16), fc_b.reshape(O, 1))
    return (logT.T, hT, cT)


# speedup vs baseline: 1.0467x; 1.0467x over previous
import jax
import jax.numpy as jnp
from jax.experimental import pallas as pl
from jax.experimental.pallas import tpu as pltpu
from jax.experimental.pallas import tpu_sc as plsc

B, T, V, D, H, O = 128, 50, 1000, 128, 256, 1000
_GATHER_WINDOW = 128
_BT = 64
_UNROLL = 10
_NS = T // _UNROLL
_F32 = jnp.float32
_BF16 = jnp.bfloat16


def _sc_gather(table, idx_flat):
    n = idx_flat.shape[0]
    idx2 = idx_flat.reshape(1, n)
    mesh = plsc.VectorSubcoreMesh(core_axis_name="core", subcore_axis_name="subcore")

    @pl.kernel(out_type=jax.ShapeDtypeStruct((n, D), table.dtype), mesh=mesh)
    def gather_kernel(tab_hbm, i_hbm, o_hbm):
        def body(i_vmem, o_vmem):
            pltpu.sync_copy(tab_hbm.at[i_vmem.at[0]], o_vmem)

        pltpu.emit_pipeline(
            body,
            grid=(n // _GATHER_WINDOW,),
            in_specs=[pl.BlockSpec((1, _GATHER_WINDOW), index_map=lambda i: (0, i))],
            out_specs=[pl.BlockSpec((_GATHER_WINDOW, D), index_map=lambda i: (i, 0))],
            core_axis_name=("core", "subcore"),
            dimension_semantics=(pltpu.PARALLEL,),
        )(i_hbm, o_hbm)

    return gather_kernel(table, idx2)


def _dot_t(x, w):
    return jax.lax.dot_general(x, w, (((1,), (1,)), ((), ())),
                               preferred_element_type=_F32)


def _body(emb_ref, hp_ref, cp_ref, wi0_ref, wh0_ref, b0_ref,
          wi1_ref, wh1_ref, b1_ref,
          out1_ref, hT_ref, cT_ref, h0s, c0s, h1s, c1s):
    t = pl.program_id(0)

    @pl.when(t == 0)
    def _():
        h0s[...] = hp_ref[0]
        c0s[...] = cp_ref[0]
        h1s[...] = hp_ref[1]
        c1s[...] = cp_ref[1]

    def gates_apply(g, cs, hs):
        i = jax.nn.sigmoid(g[:, 0 * H:1 * H])
        f = jax.nn.sigmoid(g[:, 1 * H:2 * H])
        gg = jnp.tanh(g[:, 2 * H:3 * H])
        o = jax.nn.sigmoid(g[:, 3 * H:4 * H])
        c_new = f * cs[...] + i * gg
        h_new = o * jnp.tanh(c_new)
        cs[...] = c_new
        hs[...] = h_new
        return h_new

    xb = emb_ref[...].reshape(_UNROLL * B, D).astype(_BF16)
    g0all = _dot_t(xb, wi0_ref[...]) + b0_ref[...]

    for k in range(_UNROLL):
        g0 = g0all[k * B:(k + 1) * B]
        g0 = g0 + _dot_t(h0s[...].astype(_BF16), wh0_ref[...])
        h0n = gates_apply(g0, c0s, h0s)
        g1 = _dot_t(h0n.astype(_BF16), wi1_ref[...])
        g1 = g1 + _dot_t(h1s[...].astype(_BF16), wh1_ref[...])
        g1 = g1 + b1_ref[...]
        out1_ref[k] = gates_apply(g1, c1s, h1s).astype(_BF16)

    @pl.when(t == _NS - 1)
    def _():
        hT_ref[0] = h0s[...]
        hT_ref[1] = h1s[...]
        cT_ref[0] = c0s[...]
        cT_ref[1] = c1s[...]


def _lstm_scan(emb_tb, h_prev, c_prev, wi0, wh0, b0, wi1, wh1, b1):
    full = lambda shape: pl.BlockSpec(shape, lambda t: tuple(0 for _ in shape))
    return pl.pallas_call(
        _body,
        grid=(_NS,),
        in_specs=[
            pl.BlockSpec((_UNROLL, B, D), lambda t: (t, 0, 0)),
            full((2, B, H)),
            full((2, B, H)),
            full((4 * H, D)),
            full((4 * H, H)),
            full((1, 4 * H)),
            full((4 * H, H)),
            full((4 * H, H)),
            full((1, 4 * H)),
        ],
        out_specs=[
            pl.BlockSpec((_UNROLL, B, H), lambda t: (t, 0, 0)),
            full((2, B, H)),
            full((2, B, H)),
        ],
        out_shape=[
            jax.ShapeDtypeStruct((T, B, H), _BF16),
            jax.ShapeDtypeStruct((2, B, H), _F32),
            jax.ShapeDtypeStruct((2, B, H), _F32),
        ],
        scratch_shapes=[pltpu.VMEM((B, H), _F32) for _ in range(4)],
    )(emb_tb, h_prev, c_prev, wi0, wh0, b0, wi1, wh1, b1)


def _fc_body(out1_ref, fw_ref, fb_ref, logT_ref):
    y = jnp.swapaxes(out1_ref[...], 0, 1)
    y2 = y.reshape(_BT * T, H)
    z = jax.lax.dot_general(fw_ref[...], y2, (((1,), (1,)), ((), ())),
                            preferred_element_type=_F32)
    logT_ref[...] = z + fb_ref[...]


def _fc(out1, fw, fb):
    return pl.pallas_call(
        _fc_body,
        grid=(B // _BT,),
        in_specs=[
            pl.BlockSpec((T, _BT, H), lambda i: (0, i, 0)),
            pl.BlockSpec((O, H), lambda i: (0, 0)),
            pl.BlockSpec((O, 1), lambda i: (0, 0)),
        ],
        out_specs=pl.BlockSpec((O, _BT * T), lambda i: (0, i)),
        out_shape=jax.ShapeDtypeStruct((O, B * T), _F32),
    )(out1, fw, fb)


def kernel(x, h_prev, c_prev, emb_table, W_ih0, W_hh0, b_ih0, b_hh0,
           W_ih1, W_hh1, b_ih1, b_hh1, fc_W, fc_b):
    idx = x.T.reshape(-1).astype(jnp.int32)
    emb = _sc_gather(emb_table, idx)
    emb_tb = emb.reshape(T, B, D)
    b0 = (b_ih0 + b_hh0).reshape(1, 4 * H)
    b1 = (b_ih1 + b_hh1).reshape(1, 4 * H)
    out1, hT, cT = _lstm_scan(emb_tb, h_prev, c_prev,
                              W_ih0.astype(_BF16), W_hh0.astype(_BF16), b0,
                              W_ih1.astype(_BF16), W_hh1.astype(_BF16), b1)
    logT = _fc(out1, fc_W.astype(_BF16), fc_b.reshape(O, 1))
    return (logT.T, hT, cT)
